# baseline (speedup 1.0000x reference)
import jax
import jax.numpy as jnp
from jax import lax
from jax.experimental import pallas as pl
from jax.experimental.pallas import tpu as pltpu

_INITIAL_VOCAB = 100000
_GRID = 125
_LANES = 256


def _body(idx_smem, emb_in, usage_in, ctime_in, nemb4,
          emb_out, usage_out, ctime_out, row_out):
    i = pl.program_id(0)
    tok = idx_smem[0]

    eb = emb_in.shape[0]
    exp_row = tok - _INITIAL_VOCAB
    sub = exp_row % 4
    fr = exp_row // 4 - i * eb
    rows = lax.broadcasted_iota(jnp.int32, emb_in.shape, 0)
    cols = lax.broadcasted_iota(jnp.int32, emb_in.shape, 1)
    hit = (rows == fr) & (cols >= sub * 64) & (cols < sub * 64 + 64)
    emb_out[...] = jnp.where(hit, nemb4[...], emb_in[...])

    _, ub, lanes = usage_in.shape
    ur = tok // lanes - i * ub
    uc = tok % lanes
    r2 = lax.broadcasted_iota(jnp.int32, usage_in.shape, 1)
    c2 = lax.broadcasted_iota(jnp.int32, usage_in.shape, 2)
    uhit = (r2 == ur) & (c2 == uc)
    u = usage_in[...]
    usage_out[...] = jnp.where(uhit, u + 1.0, u)

    ctime_out[...] = jnp.where(uhit, jnp.float32(jnp.inf), ctime_in[...])

    @pl.when(i == 0)
    def _():
        row_out[...] = nemb4[...]


def kernel(token_usage, token_creation_time, expanded_embeddings,
           new_embedding, new_token_id):
    idx = jnp.asarray(new_token_id, jnp.int32).reshape(1)
    n_rows, dim = expanded_embeddings.shape
    flat = expanded_embeddings.reshape(-1, _LANES)
    eb = flat.shape[0] // _GRID
    nemb4 = jnp.tile(new_embedding, _LANES // dim).reshape(1, _LANES)
    usage2 = token_usage.reshape(_GRID, -1, 64)
    ctime2 = token_creation_time.reshape(_GRID, -1, 64)
    ub = usage2.shape[1]

    expanded, usage, ctime, row = pl.pallas_call(
        _body,
        grid=(_GRID,),
        in_specs=[
            pl.BlockSpec(memory_space=pltpu.SMEM),
            pl.BlockSpec((eb, _LANES), lambda i: (i, 0)),
            pl.BlockSpec((1, ub, 64), lambda i: (i, 0, 0)),
            pl.BlockSpec((1, ub, 64), lambda i: (i, 0, 0)),
            pl.BlockSpec((1, _LANES), lambda i: (0, 0)),
        ],
        out_specs=[
            pl.BlockSpec((eb, _LANES), lambda i: (i, 0)),
            pl.BlockSpec((1, ub, 64), lambda i: (i, 0, 0)),
            pl.BlockSpec((1, ub, 64), lambda i: (i, 0, 0)),
            pl.BlockSpec((1, _LANES), lambda i: (0, 0)),
        ],
        out_shape=[
            jax.ShapeDtypeStruct(flat.shape, jnp.float32),
            jax.ShapeDtypeStruct(usage2.shape, jnp.float32),
            jax.ShapeDtypeStruct(ctime2.shape, jnp.float32),
            jax.ShapeDtypeStruct((1, _LANES), jnp.float32),
        ],
    )(idx, flat, usage2, ctime2, nemb4)
    return (row.reshape(-1)[:dim], expanded.reshape(n_rows, dim),
            usage.reshape(-1), ctime.reshape(-1))


# speedup vs baseline: 1.3013x; 1.3013x over previous
import jax
import jax.numpy as jnp
from jax import lax
from jax.experimental import pallas as pl
from jax.experimental.pallas import tpu as pltpu

_INITIAL_VOCAB = 100000
_CH = 3600
_NCH = 250
_K = 12


def _body(idx_smem, emb_in, usage_in, ctime_in, nemb_in,
          emb_out, usage_out, ctime_out, row_out,
          bufs, ubuf, cbuf, nbuf,
          sem_in, sem_out, sem_u, sem_c, sem_n):
    tok = idx_smem[0]
    exp_row = tok - _INITIAL_VOCAB

    u_in = pltpu.make_async_copy(usage_in, ubuf, sem_u)
    u_in.start()
    c_in = pltpu.make_async_copy(ctime_in, cbuf, sem_c)
    c_in.start()
    n_in = pltpu.make_async_copy(nemb_in, nbuf, sem_n)
    n_in.start()

    in_cps = {}
    out_cps = {}
    for c in range(_K):
        cp = pltpu.make_async_copy(
            emb_in.at[pl.ds(c * _CH, _CH)], bufs.at[c % _K],
            sem_in.at[c % _K])
        cp.start()
        in_cps[c] = cp

    n_in.wait()
    row_cp = pltpu.make_async_copy(nbuf, row_out, sem_n)
    row_cp.start()

    lane = lax.broadcasted_iota(jnp.int32, (1, 64), 1)

    for c in range(_NCH):
        q = c % _K
        in_cps[c].wait()

        local = exp_row - c * _CH

        @pl.when((local >= 0) & (local < _CH))
        def _(q=q, local=local):
            bufs.at[q][pl.ds(local, 1), :] = nbuf[...]

        cp = pltpu.make_async_copy(
            bufs.at[q], emb_out.at[pl.ds(c * _CH, _CH)], sem_out.at[q])
        cp.start()
        out_cps[c] = cp
        nxt = c + _K
        if nxt < _NCH:
            out_cps[c].wait()
            cp2 = pltpu.make_async_copy(
                emb_in.at[pl.ds(nxt * _CH, _CH)], bufs.at[q], sem_in.at[q])
            cp2.start()
            in_cps[nxt] = cp2

    ur = tok // 64
    uc = tok % 64
    u_in.wait()
    urow = ubuf[pl.ds(ur, 1), :]
    ubuf[pl.ds(ur, 1), :] = urow + (lane == uc).astype(jnp.float32)
    u_out = pltpu.make_async_copy(ubuf, usage_out, sem_u)
    u_out.start()

    c_in.wait()
    crow = cbuf[pl.ds(ur, 1), :]
    cbuf[pl.ds(ur, 1), :] = jnp.where(lane == uc, jnp.float32(jnp.inf), crow)
    c_out = pltpu.make_async_copy(cbuf, ctime_out, sem_c)
    c_out.start()

    for c in range(_NCH - _K, _NCH):
        out_cps[c].wait()
    row_cp.wait()
    u_out.wait()
    c_out.wait()


def kernel(token_usage, token_creation_time, expanded_embeddings,
           new_embedding, new_token_id):
    idx = jnp.asarray(new_token_id, jnp.int32).reshape(1)
    n_rows, dim = expanded_embeddings.shape
    usage2 = token_usage.reshape(-1, 64)
    ctime2 = token_creation_time.reshape(-1, 64)

    expanded, usage, ctime, row = pl.pallas_call(
        _body,
        in_specs=[
            pl.BlockSpec(memory_space=pltpu.SMEM),
            pl.BlockSpec(memory_space=pl.ANY),
            pl.BlockSpec(memory_space=pl.ANY),
            pl.BlockSpec(memory_space=pl.ANY),
            pl.BlockSpec(memory_space=pl.ANY),
        ],
        out_specs=[
            pl.BlockSpec(memory_space=pl.ANY),
            pl.BlockSpec(memory_space=pl.ANY),
            pl.BlockSpec(memory_space=pl.ANY),
            pl.BlockSpec(memory_space=pl.ANY),
        ],
        out_shape=[
            jax.ShapeDtypeStruct((n_rows, dim), jnp.float32),
            jax.ShapeDtypeStruct(usage2.shape, jnp.float32),
            jax.ShapeDtypeStruct(ctime2.shape, jnp.float32),
            jax.ShapeDtypeStruct((1, dim), jnp.float32),
        ],
        scratch_shapes=[
            pltpu.VMEM((_K, _CH, 64), jnp.float32),
            pltpu.VMEM(usage2.shape, jnp.float32),
            pltpu.VMEM(ctime2.shape, jnp.float32),
            pltpu.VMEM((1, 64), jnp.float32),
            pltpu.SemaphoreType.DMA((_K,)),
            pltpu.SemaphoreType.DMA((_K,)),
            pltpu.SemaphoreType.DMA,
            pltpu.SemaphoreType.DMA,
            pltpu.SemaphoreType.DMA,
        ],
    )(idx, expanded_embeddings, usage2, ctime2, new_embedding.reshape(1, -1))
    return (row.reshape(-1), expanded, usage.reshape(-1), ctime.reshape(-1))
